# baseline (speedup 1.0000x reference)
import functools

import jax
import jax.numpy as jnp
from jax import lax
from jax.experimental import pallas as pl
from jax.experimental.pallas import tpu as pltpu
from jax.experimental.pallas import tpu_sc as plsc

OUT_DIM = 64
PAD_W = 128
SEQ = 200
PAD_IDX = 0
NUM_CORES = 2
NUM_SUBCORES = 16
NUM_WORKERS = NUM_CORES * NUM_SUBCORES
CHUNK_B = 4
CHUNK = CHUNK_B * SEQ


def _emb_body(idx_hbm, table_hbm, out_hbm, idx_v, rows_v, sem, *, b_per_w):
  wid = lax.axis_index("s") * NUM_CORES + lax.axis_index("c")
  base = wid * b_per_w
  nchunk = b_per_w // CHUNK

  def chunk_body(k, carry):
    off = base + k * CHUNK
    pltpu.sync_copy(idx_hbm.at[pl.ds(off, CHUNK)], idx_v)
    pltpu.async_copy(table_hbm.at[idx_v], rows_v, sem).wait()

    def grp(g, tot):
      v = idx_v[pl.ds(g * 16, 16)]
      return tot + jnp.sum((v == PAD_IDX).astype(jnp.int32))

    tot = lax.fori_loop(0, CHUNK // 16, grp, jnp.int32(0))

    @pl.when(tot > 0)
    def _():
      zeros = jnp.zeros((16,), jnp.float32)

      def fix_grp(g, c):
        v = idx_v[pl.ds(g * 16, 16)]
        cnt = jnp.sum((v == PAD_IDX).astype(jnp.int32))

        @pl.when(cnt > 0)
        def _():
          for j in range(16):
            @pl.when(v[j] == PAD_IDX)
            def _():
              for q in range(OUT_DIM // 16):
                rows_v[g * 16 + j, pl.ds(q * 16, 16)] = zeros

        return c

      lax.fori_loop(0, CHUNK // 16, fix_grp, 0)

    bent = off // SEQ
    for e in range(CHUNK_B):
      pltpu.sync_copy(rows_v.at[pl.ds(e * SEQ, SEQ)],
                      out_hbm.at[bent + e, :, pl.ds(0, OUT_DIM)])
    return carry

  lax.fori_loop(0, nchunk, chunk_body, 0)


def kernel(inputs, embeddings):
  b, l = inputs.shape
  n = b * l
  assert n % NUM_WORKERS == 0
  b_per_w = n // NUM_WORKERS
  assert b_per_w % CHUNK == 0
  v = embeddings.shape[0]

  idx = inputs.reshape(n).astype(jnp.int32)
  emb2 = jnp.concatenate([embeddings[0::2], embeddings[1::2]], axis=1)
  emb2 = lax.optimization_barrier(emb2)
  emb_lin = emb2.reshape(v, OUT_DIM)

  mesh = plsc.VectorSubcoreMesh(
      core_axis_name="c", subcore_axis_name="s", num_cores=NUM_CORES,
      num_subcores=NUM_SUBCORES)
  fn = pl.kernel(
      functools.partial(_emb_body, b_per_w=b_per_w),
      out_type=jax.ShapeDtypeStruct((b, l, PAD_W), jnp.float32),
      mesh=mesh,
      scratch_types=[
          pltpu.VMEM((CHUNK,), jnp.int32),
          pltpu.VMEM((CHUNK, OUT_DIM), jnp.float32),
          pltpu.SemaphoreType.DMA,
      ],
      compiler_params=pltpu.CompilerParams(
          use_tc_tiling_on_sc=False, needs_layout_passes=False),
  )
  return fn(idx, emb_lin)[:, :, :OUT_DIM]


# speedup vs baseline: 9.7937x; 9.7937x over previous
import functools

import jax
import jax.numpy as jnp
from jax import lax
from jax.experimental import pallas as pl
from jax.experimental.pallas import tpu as pltpu
from jax.experimental.pallas import tpu_sc as plsc

OUT_DIM = 64
PAD_W = 128
SEQ = 200
PAD_IDX = 0
NUM_CORES = 2
NUM_SUBCORES = 16
NUM_WORKERS = NUM_CORES * NUM_SUBCORES
CHUNK_B = 4
CHUNK = CHUNK_B * SEQ


def _emb_body(idx_hbm, table_hbm, out_hbm,
              idx_v0, idx_v1, rows_v0, rows_v1,
              gsem0, gsem1, wsem0, wsem1, *, b_per_w):
  wid = lax.axis_index("s") * NUM_CORES + lax.axis_index("c")
  base = wid * b_per_w
  nchunk = b_per_w // CHUNK
  idx_v = (idx_v0, idx_v1)
  rows_v = (rows_v0, rows_v1)
  gsem = (gsem0, gsem1)
  wsem = (wsem0, wsem1)

  def fixup(iv, rv):
    def grp(g, tot):
      v = iv[pl.ds(g * 16, 16)]
      return tot + jnp.sum((v == PAD_IDX).astype(jnp.int32))

    tot = lax.fori_loop(0, CHUNK // 16, grp, jnp.int32(0))

    @pl.when(tot > 0)
    def _():
      zeros = jnp.zeros((16,), jnp.float32)

      def fix_grp(g, c):
        v = iv[pl.ds(g * 16, 16)]
        cnt = jnp.sum((v == PAD_IDX).astype(jnp.int32))

        @pl.when(cnt > 0)
        def _():
          for j in range(16):
            @pl.when(v[j] == PAD_IDX)
            def _():
              for q in range(OUT_DIM // 16):
                rv[g * 16 + j, pl.ds(q * 16, 16)] = zeros

        return c

      lax.fori_loop(0, CHUNK // 16, fix_grp, 0)

  def start_writes(k, cur):
    bent = (base + k * CHUNK) // SEQ
    for e in range(CHUNK_B):
      pltpu.async_copy(rows_v[cur].at[pl.ds(e * SEQ, SEQ)],
                       out_hbm.at[bent + e, :, pl.ds(0, OUT_DIM)],
                       wsem[cur])

  def drain_writes(k, cur):
    bent = (base + k * CHUNK) // SEQ
    for e in range(CHUNK_B):
      pltpu.make_async_copy(rows_v[cur].at[pl.ds(e * SEQ, SEQ)],
                            out_hbm.at[bent + e, :, pl.ds(0, OUT_DIM)],
                            wsem[cur]).wait()

  pltpu.sync_copy(idx_hbm.at[pl.ds(base, CHUNK)], idx_v[0])
  pltpu.async_copy(table_hbm.at[idx_v[0]], rows_v[0], gsem[0])

  def pair_body(k2, carry):
    for h in range(2):
      cur, oth = h, 1 - h
      k = k2 * 2 + h
      pltpu.make_async_copy(table_hbm.at[idx_v[cur]], rows_v[cur],
                            gsem[cur]).wait()
      fixup(idx_v[cur], rows_v[cur])
      start_writes(k, cur)

      @pl.when(k + 1 < nchunk)
      def _():
        @pl.when(k >= 1)
        def _():
          drain_writes(k - 1, oth)

        off = base + (k + 1) * CHUNK
        pltpu.sync_copy(idx_hbm.at[pl.ds(off, CHUNK)], idx_v[oth])
        pltpu.async_copy(table_hbm.at[idx_v[oth]], rows_v[oth], gsem[oth])

    return carry

  lax.fori_loop(0, nchunk // 2, pair_body, 0)
  drain_writes(nchunk - 2, 0)
  drain_writes(nchunk - 1, 1)


def kernel(inputs, embeddings):
  b, l = inputs.shape
  n = b * l
  assert n % NUM_WORKERS == 0
  b_per_w = n // NUM_WORKERS
  assert b_per_w % CHUNK == 0 and (b_per_w // CHUNK) % 2 == 0
  v = embeddings.shape[0]

  idx = inputs.reshape(n).astype(jnp.int32)
  emb2 = embeddings.reshape(v // 2, 2 * OUT_DIM)
  emb2 = lax.optimization_barrier(emb2)
  emb_lin = emb2.reshape(v, OUT_DIM)

  mesh = plsc.VectorSubcoreMesh(
      core_axis_name="c", subcore_axis_name="s", num_cores=NUM_CORES,
      num_subcores=NUM_SUBCORES)
  fn = pl.kernel(
      functools.partial(_emb_body, b_per_w=b_per_w),
      out_type=jax.ShapeDtypeStruct((b, l, PAD_W), jnp.float32),
      mesh=mesh,
      scratch_types=[
          pltpu.VMEM((CHUNK,), jnp.int32),
          pltpu.VMEM((CHUNK,), jnp.int32),
          pltpu.VMEM((CHUNK, OUT_DIM), jnp.float32),
          pltpu.VMEM((CHUNK, OUT_DIM), jnp.float32),
          pltpu.SemaphoreType.DMA,
          pltpu.SemaphoreType.DMA,
          pltpu.SemaphoreType.DMA,
          pltpu.SemaphoreType.DMA,
      ],
      compiler_params=pltpu.CompilerParams(
          use_tc_tiling_on_sc=False, needs_layout_passes=False),
  )
  return fn(idx, emb_lin)[:, :, :OUT_DIM]
